# Initial kernel scaffold; baseline (speedup 1.0000x reference)
import jax
import jax.numpy as jnp
from jax.experimental import pallas as pl


def kernel(x, edge_index, W1, proj_W, proj_b, bias):
    raise NotImplementedError("write your pallas kernel here")


# speedup vs baseline: 7.0529x; 7.0529x over previous
import functools

import jax
import jax.numpy as jnp
from jax import lax
from jax.experimental import pallas as pl
from jax.experimental.pallas import tpu as pltpu
from jax.experimental.pallas import tpu_sc as plsc

N = 10000
E = 320000
D = 128

NC = 2
NS = 16
NW = NC * NS
EPW = 10240
E_PAD = NW * EPW
K = 80
CHUNKS = EPW // K
NPAD = 10240
RPW = NPAD // NS


def _tc_front(x_ref, w1_ref, wa_ref, wb_ref, pb_ref, h_ref, a_ref, b_ref):
    h = jnp.dot(x_ref[...], w1_ref[...], preferred_element_type=jnp.float32)
    h_ref[...] = h
    a_ref[...] = jnp.dot(h, wa_ref[...], preferred_element_type=jnp.float32) + pb_ref[0, 0]
    b_ref[...] = jnp.dot(h, wb_ref[...], preferred_element_type=jnp.float32)


def _tc_back(p0_ref, p1_ref, d0_ref, d1_ref, bias_ref, o_ref):
    den = d0_ref[...] + d1_ref[...] + 1e-16
    o_ref[...] = (p0_ref[...] + p1_ref[...]) / den + bias_ref[...]


def _sc_body(h_hbm, a_hbm, b_hbm, pk_hbm, out_hbm, den_hbm,
             a_v, b_v, pk_v, src_cur, dst_cur, num_cur, rows, sh_out,
             sh_den, sem):
    cid = lax.axis_index("c")
    sid = lax.axis_index("s")
    wid = sid * NC + cid

    pltpu.sync_copy(a_hbm, a_v)
    pltpu.sync_copy(b_hbm, b_v)
    pltpu.sync_copy(pk_hbm.at[pl.ds(wid * EPW, EPW)], pk_v)

    zf = jnp.zeros((16,), jnp.float32)

    def _zero_rows(i, c):
        for d in range(8):
            rows[i, pl.ds(d * 16, 16)] = zf
        return c

    lax.fori_loop(0, K, _zero_rows, 0)
    for g in range(K // 16):
        num_cur[pl.ds(g * 16, 16)] = zf
    base = sid * RPW
    for k in range(RPW // K):
        pltpu.sync_copy(rows, sh_out.at[pl.ds(base + k * K, K)])
        pltpu.sync_copy(num_cur, sh_den.at[pl.ds(base + k * K, K)])
    plsc.subcore_barrier()

    def _chunk(j, c):
        for g in range(K // 16):
            sl = pl.ds(g * 16, 16)
            p = pk_v[pl.ds(j * K + g * 16, 16)]
            src_cur[sl] = p & 0xFFFF
            dst_cur[sl] = lax.shift_right_logical(p, 16)
        cp = pltpu.async_copy(h_hbm.at[src_cur], rows, sem)
        for g in range(K // 16):
            sl = pl.ds(g * 16, 16)
            av = plsc.load_gather(a_v, [dst_cur[sl]])
            bv = plsc.load_gather(b_v, [src_cur[sl]])
            v = av + bv
            v = jnp.where(v >= 0.0, v, 0.2 * v)
            num_cur[sl] = jnp.exp(v)
        pltpu.sync_copy(num_cur, sh_den.at[dst_cur], add=True)
        cp.wait()

        def _scale(e, cc):
            s = plsc.load_gather(num_cur, [jnp.full((16,), e, jnp.int32)])
            for d in range(8):
                sl = pl.ds(d * 16, 16)
                rows[e, sl] = rows[e, sl] * s
            return cc

        lax.fori_loop(0, K, _scale, 0)
        pltpu.sync_copy(rows, sh_out.at[dst_cur], add=True)
        return c

    lax.fori_loop(0, CHUNKS, _chunk, 0)
    plsc.subcore_barrier()

    pltpu.sync_copy(sh_out.at[pl.ds(base, RPW)], out_hbm.at[cid, pl.ds(base, RPW)])
    pltpu.sync_copy(sh_den.at[pl.ds(base, RPW)], den_hbm.at[cid, pl.ds(base, RPW)])


_sc_kernel = functools.partial(
    pl.kernel,
    out_type=[
        jax.ShapeDtypeStruct((NC, NPAD, D), jnp.float32),
        jax.ShapeDtypeStruct((NC, NPAD), jnp.float32),
    ],
    mesh=plsc.VectorSubcoreMesh(core_axis_name="c", subcore_axis_name="s"),
    compiler_params=pltpu.CompilerParams(needs_layout_passes=False),
    scratch_types=[
        pltpu.VMEM((NPAD,), jnp.float32),
        pltpu.VMEM((NPAD,), jnp.float32),
        pltpu.VMEM((EPW,), jnp.int32),
        pltpu.VMEM((K,), jnp.int32),
        pltpu.VMEM((K,), jnp.int32),
        pltpu.VMEM((K,), jnp.float32),
        pltpu.VMEM((K, D), jnp.float32),
        pltpu.VMEM_SHARED((NPAD, D), jnp.float32),
        pltpu.VMEM_SHARED((NPAD,), jnp.float32),
        pltpu.SemaphoreType.DMA,
    ],
)(_sc_body)


@jax.jit
def kernel(x, edge_index, W1, proj_W, proj_b, bias):
    src = edge_index[0].astype(jnp.int32)
    dst = edge_index[1].astype(jnp.int32)
    packed = jnp.bitwise_or(jnp.left_shift(dst, 16), src)
    pad_val = jnp.int32((NPAD - 1) << 16)
    packed = jnp.concatenate(
        [packed, jnp.full((E_PAD - E,), pad_val, jnp.int32)])
    wa = proj_W[:D]
    wb = proj_W[D:]
    pb = proj_b.reshape(1, 1)

    rb = 1000
    grid_f = N // rb
    h, a, b = pl.pallas_call(
        _tc_front,
        grid=(grid_f,),
        in_specs=[
            pl.BlockSpec((rb, D), lambda i: (i, 0)),
            pl.BlockSpec((D, D), lambda i: (0, 0)),
            pl.BlockSpec((D, 1), lambda i: (0, 0)),
            pl.BlockSpec((D, 1), lambda i: (0, 0)),
            pl.BlockSpec(memory_space=pltpu.SMEM),
        ],
        out_specs=[
            pl.BlockSpec((rb, D), lambda i: (i, 0)),
            pl.BlockSpec((rb, 1), lambda i: (i, 0)),
            pl.BlockSpec((rb, 1), lambda i: (i, 0)),
        ],
        out_shape=[
            jax.ShapeDtypeStruct((N, D), jnp.float32),
            jax.ShapeDtypeStruct((N, 1), jnp.float32),
            jax.ShapeDtypeStruct((N, 1), jnp.float32),
        ],
    )(x, W1, wa, wb, pb)

    zpad = jnp.zeros((NPAD - N,), jnp.float32)
    a_p = jnp.concatenate([a.reshape(N), zpad])
    b_p = jnp.concatenate([b.reshape(N), zpad])

    out_part, den_part = _sc_kernel(h, a_p, b_p, packed)

    rb2 = 1024
    grid_b = NPAD // rb2
    out_full = pl.pallas_call(
        _tc_back,
        grid=(grid_b,),
        in_specs=[
            pl.BlockSpec((rb2, D), lambda i: (i, 0)),
            pl.BlockSpec((rb2, D), lambda i: (i, 0)),
            pl.BlockSpec((rb2, 1), lambda i: (i, 0)),
            pl.BlockSpec((rb2, 1), lambda i: (i, 0)),
            pl.BlockSpec((1, D), lambda i: (0, 0)),
        ],
        out_specs=pl.BlockSpec((rb2, D), lambda i: (i, 0)),
        out_shape=jax.ShapeDtypeStruct((NPAD, D), jnp.float32),
    )(out_part[0], out_part[1], den_part[0].reshape(NPAD, 1),
      den_part[1].reshape(NPAD, 1), bias.reshape(1, D))

    return out_full[:N]


# speedup vs baseline: 9.4101x; 1.3342x over previous
import functools

import jax
import jax.numpy as jnp
from jax import lax
from jax.experimental import pallas as pl
from jax.experimental.pallas import tpu as pltpu
from jax.experimental.pallas import tpu_sc as plsc

N = 10000
E = 320000
D = 128

NC = 2
NS = 16
NW = NC * NS
EPW = 10240
E_PAD = NW * EPW
K = 64
CHUNKS = EPW // K
NPAD = 10240
RPW = NPAD // NS


def _tc_front(x_ref, w1_ref, wa_ref, wb_ref, pb_ref, h_ref, a_ref, b_ref):
    h = jnp.dot(x_ref[...], w1_ref[...], preferred_element_type=jnp.float32)
    h_ref[...] = h
    a_ref[...] = jnp.dot(h, wa_ref[...], preferred_element_type=jnp.float32) + pb_ref[0, 0]
    b_ref[...] = jnp.dot(h, wb_ref[...], preferred_element_type=jnp.float32)


def _tc_back(p0_ref, p1_ref, d0_ref, d1_ref, bias_ref, o_ref):
    den = d0_ref[...] + d1_ref[...] + 1e-16
    o_ref[...] = (p0_ref[...] + p1_ref[...]) / den + bias_ref[...]


def _sc_body(h_hbm, a_hbm, b_hbm, pk_hbm, out_hbm, den_hbm,
             a_v, b_v, pk_v, src0, dst0, src1, dst1, num0, num1,
             rows0, rows1, sh_out, sh_den, sem0, sem1):
    cid = lax.axis_index("c")
    sid = lax.axis_index("s")
    wid = sid * NC + cid

    pltpu.sync_copy(a_hbm, a_v)
    pltpu.sync_copy(b_hbm, b_v)
    pltpu.sync_copy(pk_hbm.at[pl.ds(wid * EPW, EPW)], pk_v)

    zf = jnp.zeros((16,), jnp.float32)

    def _zero_rows(i, c):
        for d in range(8):
            rows0[i, pl.ds(d * 16, 16)] = zf
        return c

    lax.fori_loop(0, K, _zero_rows, 0)
    for g in range(K // 16):
        num0[pl.ds(g * 16, 16)] = zf
    base = sid * RPW
    for k in range(RPW // K):
        pltpu.sync_copy(rows0, sh_out.at[pl.ds(base + k * K, K)])
        pltpu.sync_copy(num0, sh_den.at[pl.ds(base + k * K, K)])
    plsc.subcore_barrier()

    def _unpack(j, src_c, dst_c):
        for g in range(K // 16):
            sl = pl.ds(g * 16, 16)
            p = pk_v[pl.ds(j * K + g * 16, 16)]
            src_c[sl] = p & 0xFFFF
            dst_c[sl] = lax.shift_right_logical(p, 16)

    def _step(j, src_c, dst_c, num_c, rows_c, sem_c, src_n, dst_n, rows_n,
              sem_n):
        @pl.when(j + 1 < CHUNKS)
        def _():
            _unpack(j + 1, src_n, dst_n)
            pltpu.async_copy(h_hbm.at[src_n], rows_n, sem_n)

        for g in range(K // 16):
            sl = pl.ds(g * 16, 16)
            av = plsc.load_gather(a_v, [dst_c[sl]])
            bv = plsc.load_gather(b_v, [src_c[sl]])
            v = av + bv
            v = jnp.where(v >= 0.0, v, 0.2 * v)
            num_c[sl] = jnp.exp(v)
        pltpu.sync_copy(num_c, sh_den.at[dst_c], add=True)
        pltpu.make_async_copy(h_hbm.at[src_c], rows_c, sem_c).wait()

        def _sgrp(g, cc):
            for e16 in range(16):
                e = g * 16 + e16
                s = plsc.load_gather(num_c, [jnp.full((16,), e, jnp.int32)])
                for d in range(8):
                    sl = pl.ds(d * 16, 16)
                    rows_c[e, sl] = rows_c[e, sl] * s
            return cc

        lax.fori_loop(0, K // 16, _sgrp, 0)
        pltpu.sync_copy(rows_c, sh_out.at[dst_c], add=True)

    _unpack(0, src0, dst0)
    pltpu.async_copy(h_hbm.at[src0], rows0, sem0)

    def _pair(j2, c):
        j = j2 * 2
        _step(j, src0, dst0, num0, rows0, sem0, src1, dst1, rows1, sem1)
        _step(j + 1, src1, dst1, num1, rows1, sem1, src0, dst0, rows0, sem0)
        return c

    lax.fori_loop(0, CHUNKS // 2, _pair, 0)
    plsc.subcore_barrier()

    pltpu.sync_copy(sh_out.at[pl.ds(base, RPW)], out_hbm.at[cid, pl.ds(base, RPW)])
    pltpu.sync_copy(sh_den.at[pl.ds(base, RPW)], den_hbm.at[cid, pl.ds(base, RPW)])


_sc_kernel = functools.partial(
    pl.kernel,
    out_type=[
        jax.ShapeDtypeStruct((NC, NPAD, D), jnp.float32),
        jax.ShapeDtypeStruct((NC, NPAD), jnp.float32),
    ],
    mesh=plsc.VectorSubcoreMesh(core_axis_name="c", subcore_axis_name="s"),
    compiler_params=pltpu.CompilerParams(needs_layout_passes=False),
    scratch_types=[
        pltpu.VMEM((NPAD,), jnp.float32),
        pltpu.VMEM((NPAD,), jnp.float32),
        pltpu.VMEM((EPW,), jnp.int32),
        pltpu.VMEM((K,), jnp.int32),
        pltpu.VMEM((K,), jnp.int32),
        pltpu.VMEM((K,), jnp.int32),
        pltpu.VMEM((K,), jnp.int32),
        pltpu.VMEM((K,), jnp.float32),
        pltpu.VMEM((K,), jnp.float32),
        pltpu.VMEM((K, D), jnp.float32),
        pltpu.VMEM((K, D), jnp.float32),
        pltpu.VMEM_SHARED((NPAD, D), jnp.float32),
        pltpu.VMEM_SHARED((NPAD,), jnp.float32),
        pltpu.SemaphoreType.DMA,
        pltpu.SemaphoreType.DMA,
    ],
)(_sc_body)


@jax.jit
def kernel(x, edge_index, W1, proj_W, proj_b, bias):
    src = edge_index[0].astype(jnp.int32)
    dst = edge_index[1].astype(jnp.int32)
    packed = jnp.bitwise_or(jnp.left_shift(dst, 16), src)
    pad_val = jnp.int32((NPAD - 1) << 16)
    packed = jnp.concatenate(
        [packed, jnp.full((E_PAD - E,), pad_val, jnp.int32)])
    wa = proj_W[:D]
    wb = proj_W[D:]
    pb = proj_b.reshape(1, 1)

    rb = 1000
    grid_f = N // rb
    h, a, b = pl.pallas_call(
        _tc_front,
        grid=(grid_f,),
        in_specs=[
            pl.BlockSpec((rb, D), lambda i: (i, 0)),
            pl.BlockSpec((D, D), lambda i: (0, 0)),
            pl.BlockSpec((D, 1), lambda i: (0, 0)),
            pl.BlockSpec((D, 1), lambda i: (0, 0)),
            pl.BlockSpec(memory_space=pltpu.SMEM),
        ],
        out_specs=[
            pl.BlockSpec((rb, D), lambda i: (i, 0)),
            pl.BlockSpec((rb, 1), lambda i: (i, 0)),
            pl.BlockSpec((rb, 1), lambda i: (i, 0)),
        ],
        out_shape=[
            jax.ShapeDtypeStruct((N, D), jnp.float32),
            jax.ShapeDtypeStruct((N, 1), jnp.float32),
            jax.ShapeDtypeStruct((N, 1), jnp.float32),
        ],
    )(x, W1, wa, wb, pb)

    zpad = jnp.zeros((NPAD - N,), jnp.float32)
    a_p = jnp.concatenate([a.reshape(N), zpad])
    b_p = jnp.concatenate([b.reshape(N), zpad])

    out_part, den_part = _sc_kernel(h, a_p, b_p, packed)

    rb2 = 1024
    grid_b = NPAD // rb2
    out_full = pl.pallas_call(
        _tc_back,
        grid=(grid_b,),
        in_specs=[
            pl.BlockSpec((rb2, D), lambda i: (i, 0)),
            pl.BlockSpec((rb2, D), lambda i: (i, 0)),
            pl.BlockSpec((rb2, 1), lambda i: (i, 0)),
            pl.BlockSpec((rb2, 1), lambda i: (i, 0)),
            pl.BlockSpec((1, D), lambda i: (0, 0)),
        ],
        out_specs=pl.BlockSpec((rb2, D), lambda i: (i, 0)),
        out_shape=jax.ShapeDtypeStruct((NPAD, D), jnp.float32),
    )(out_part[0], out_part[1], den_part[0].reshape(NPAD, 1),
      den_part[1].reshape(NPAD, 1), bias.reshape(1, D))

    return out_full[:N]


# speedup vs baseline: 9.4834x; 1.0078x over previous
import functools

import jax
import jax.numpy as jnp
from jax import lax
from jax.experimental import pallas as pl
from jax.experimental.pallas import tpu as pltpu
from jax.experimental.pallas import tpu_sc as plsc

N = 10000
E = 320000
D = 128

NC = 2
NS = 16
NW = NC * NS
EPW = 10240
E_PAD = NW * EPW
K = 64
CHUNKS = EPW // K
NPAD = 10240
RPW = NPAD // NS


def _tc_front(x_ref, w1_ref, wa_ref, wb_ref, pb_ref, h_ref, a_ref, b_ref):
    h = jnp.dot(x_ref[...], w1_ref[...], preferred_element_type=jnp.float32)
    h_ref[...] = h
    a_ref[...] = jnp.dot(h, wa_ref[...], preferred_element_type=jnp.float32) + pb_ref[0, 0]
    b_ref[...] = jnp.dot(h, wb_ref[...], preferred_element_type=jnp.float32)


def _tc_back(p0_ref, p1_ref, d0_ref, d1_ref, bias_ref, o_ref):
    den = d0_ref[...] + d1_ref[...] + 1e-16
    o_ref[...] = (p0_ref[...] + p1_ref[...]) / den + bias_ref[...]


def _sc_body(h_hbm, a_hbm, b_hbm, pk_hbm, out_hbm, den_hbm,
             a_v, b_v, pk_v, src0, dst0, src1, dst1, num0, num1,
             rows0, rows1, sh_out, sh_den, sem0, sem1, scs0, scs1):
    cid = lax.axis_index("c")
    sid = lax.axis_index("s")
    wid = sid * NC + cid

    pltpu.sync_copy(a_hbm, a_v)
    pltpu.sync_copy(b_hbm, b_v)
    pltpu.sync_copy(pk_hbm.at[pl.ds(wid * EPW, EPW)], pk_v)

    zf = jnp.zeros((16,), jnp.float32)

    def _zero_rows(i, c):
        for d in range(8):
            rows0[i, pl.ds(d * 16, 16)] = zf
        return c

    lax.fori_loop(0, K, _zero_rows, 0)
    for g in range(K // 16):
        num0[pl.ds(g * 16, 16)] = zf
    base = sid * RPW
    for k in range(RPW // K):
        pltpu.sync_copy(rows0, sh_out.at[pl.ds(base + k * K, K)])
        pltpu.sync_copy(num0, sh_den.at[pl.ds(base + k * K, K)])
    plsc.subcore_barrier()

    def _unpack(j, src_c, dst_c):
        for g in range(K // 16):
            sl = pl.ds(g * 16, 16)
            p = pk_v[pl.ds(j * K + g * 16, 16)]
            src_c[sl] = p & 0xFFFF
            dst_c[sl] = lax.shift_right_logical(p, 16)

    def _step(j, src_c, dst_c, num_c, rows_c, sem_c, scs_c,
              src_n, dst_n, num_n, rows_n, sem_n, scs_n):
        for g in range(K // 16):
            sl = pl.ds(g * 16, 16)
            av = plsc.load_gather(a_v, [dst_c[sl]])
            bv = plsc.load_gather(b_v, [src_c[sl]])
            v = av + bv
            v = jnp.where(v >= 0.0, v, 0.2 * v)
            num_c[sl] = jnp.exp(v)
        pltpu.async_copy(num_c, sh_den.at[dst_c], scs_c, add=True)

        @pl.when(j >= 1)
        def _():
            pltpu.make_async_copy(num_n, sh_den.at[dst_n], scs_n).wait()
            pltpu.make_async_copy(rows_n, sh_out.at[dst_n], scs_n).wait()

        @pl.when(j + 1 < CHUNKS)
        def _():
            _unpack(j + 1, src_n, dst_n)
            pltpu.async_copy(h_hbm.at[src_n], rows_n, sem_n)

        pltpu.make_async_copy(h_hbm.at[src_c], rows_c, sem_c).wait()

        def _sgrp(g, cc):
            for e16 in range(16):
                e = g * 16 + e16
                s = plsc.load_gather(num_c, [jnp.full((16,), e, jnp.int32)])
                for d in range(8):
                    sl = pl.ds(d * 16, 16)
                    rows_c[e, sl] = rows_c[e, sl] * s
            return cc

        lax.fori_loop(0, K // 16, _sgrp, 0)
        pltpu.async_copy(rows_c, sh_out.at[dst_c], scs_c, add=True)

    _unpack(0, src0, dst0)
    pltpu.async_copy(h_hbm.at[src0], rows0, sem0)

    def _pair(j2, c):
        j = j2 * 2
        _step(j, src0, dst0, num0, rows0, sem0, scs0,
              src1, dst1, num1, rows1, sem1, scs1)
        _step(j + 1, src1, dst1, num1, rows1, sem1, scs1,
              src0, dst0, num0, rows0, sem0, scs0)
        return c

    lax.fori_loop(0, CHUNKS // 2, _pair, 0)
    pltpu.make_async_copy(num1, sh_den.at[dst1], scs1).wait()
    pltpu.make_async_copy(rows1, sh_out.at[dst1], scs1).wait()
    plsc.subcore_barrier()

    pltpu.sync_copy(sh_out.at[pl.ds(base, RPW)], out_hbm.at[cid, pl.ds(base, RPW)])
    pltpu.sync_copy(sh_den.at[pl.ds(base, RPW)], den_hbm.at[cid, pl.ds(base, RPW)])


_sc_kernel = functools.partial(
    pl.kernel,
    out_type=[
        jax.ShapeDtypeStruct((NC, NPAD, D), jnp.float32),
        jax.ShapeDtypeStruct((NC, NPAD), jnp.float32),
    ],
    mesh=plsc.VectorSubcoreMesh(core_axis_name="c", subcore_axis_name="s"),
    compiler_params=pltpu.CompilerParams(needs_layout_passes=False),
    scratch_types=[
        pltpu.VMEM((NPAD,), jnp.float32),
        pltpu.VMEM((NPAD,), jnp.float32),
        pltpu.VMEM((EPW,), jnp.int32),
        pltpu.VMEM((K,), jnp.int32),
        pltpu.VMEM((K,), jnp.int32),
        pltpu.VMEM((K,), jnp.int32),
        pltpu.VMEM((K,), jnp.int32),
        pltpu.VMEM((K,), jnp.float32),
        pltpu.VMEM((K,), jnp.float32),
        pltpu.VMEM((K, D), jnp.float32),
        pltpu.VMEM((K, D), jnp.float32),
        pltpu.VMEM_SHARED((NPAD, D), jnp.float32),
        pltpu.VMEM_SHARED((NPAD,), jnp.float32),
        pltpu.SemaphoreType.DMA,
        pltpu.SemaphoreType.DMA,
        pltpu.SemaphoreType.DMA,
        pltpu.SemaphoreType.DMA,
    ],
)(_sc_body)


@jax.jit
def kernel(x, edge_index, W1, proj_W, proj_b, bias):
    src = edge_index[0].astype(jnp.int32)
    dst = edge_index[1].astype(jnp.int32)
    packed = jnp.bitwise_or(jnp.left_shift(dst, 16), src)
    pad_val = jnp.int32((NPAD - 1) << 16)
    packed = jnp.concatenate(
        [packed, jnp.full((E_PAD - E,), pad_val, jnp.int32)])
    wa = proj_W[:D]
    wb = proj_W[D:]
    pb = proj_b.reshape(1, 1)

    rb = 1000
    grid_f = N // rb
    h, a, b = pl.pallas_call(
        _tc_front,
        grid=(grid_f,),
        in_specs=[
            pl.BlockSpec((rb, D), lambda i: (i, 0)),
            pl.BlockSpec((D, D), lambda i: (0, 0)),
            pl.BlockSpec((D, 1), lambda i: (0, 0)),
            pl.BlockSpec((D, 1), lambda i: (0, 0)),
            pl.BlockSpec(memory_space=pltpu.SMEM),
        ],
        out_specs=[
            pl.BlockSpec((rb, D), lambda i: (i, 0)),
            pl.BlockSpec((rb, 1), lambda i: (i, 0)),
            pl.BlockSpec((rb, 1), lambda i: (i, 0)),
        ],
        out_shape=[
            jax.ShapeDtypeStruct((N, D), jnp.float32),
            jax.ShapeDtypeStruct((N, 1), jnp.float32),
            jax.ShapeDtypeStruct((N, 1), jnp.float32),
        ],
    )(x, W1, wa, wb, pb)

    zpad = jnp.zeros((NPAD - N,), jnp.float32)
    a_p = jnp.concatenate([a.reshape(N), zpad])
    b_p = jnp.concatenate([b.reshape(N), zpad])

    out_part, den_part = _sc_kernel(h, a_p, b_p, packed)

    rb2 = 1024
    grid_b = NPAD // rb2
    out_full = pl.pallas_call(
        _tc_back,
        grid=(grid_b,),
        in_specs=[
            pl.BlockSpec((rb2, D), lambda i: (i, 0)),
            pl.BlockSpec((rb2, D), lambda i: (i, 0)),
            pl.BlockSpec((rb2, 1), lambda i: (i, 0)),
            pl.BlockSpec((rb2, 1), lambda i: (i, 0)),
            pl.BlockSpec((1, D), lambda i: (0, 0)),
        ],
        out_specs=pl.BlockSpec((rb2, D), lambda i: (i, 0)),
        out_shape=jax.ShapeDtypeStruct((NPAD, D), jnp.float32),
    )(out_part[0], out_part[1], den_part[0].reshape(NPAD, 1),
      den_part[1].reshape(NPAD, 1), bias.reshape(1, D))

    return out_full[:N]
